# Initial kernel scaffold; baseline (speedup 1.0000x reference)
import functools

import jax
import jax.numpy as jnp
from jax import lax
from jax.experimental import pallas as pl
from jax.experimental.pallas import tpu as pltpu
from jax.experimental.pallas import tpu_sc as plsc

_B = 2
_V = 4
_P = 2048
_S = 49152
_C = 64
_NVAR = 8

_NW = 32
_RPW = (_B * _V * _P) // _NW
_CH = 128
_NCH = _RPW // _CH
_LANES = 16


def _make_sc_gather():
    info = plsc.get_sparse_core_info()
    nc = info.num_cores

    mesh = plsc.VectorSubcoreMesh(core_axis_name="c", subcore_axis_name="s")

    @functools.partial(
        pl.kernel,
        mesh=mesh,
        out_type=jax.ShapeDtypeStruct((_NW, _NCH, _CH, _C), jnp.float32),
        scratch_types=[
            pltpu.VMEM((_RPW,), jnp.int32),
            pltpu.VMEM((_LANES,), jnp.int32),
            pltpu.VMEM((_NCH, _CH), jnp.int32),
            pltpu.VMEM((_NCH, _CH, _C), jnp.float32),
            pltpu.SemaphoreType.DMA,
        ],
    )
    def gather_kernel(vb_hbm, patch_hbm, table_hbm, out_hbm,
                      patch_v, var_v, idx_v, rows_v, sem):
        wid = lax.axis_index("s") * nc + lax.axis_index("c")
        b = wid // (_NW // _B)
        p0 = (wid % (_P // _RPW)) * _RPW
        pltpu.sync_copy(patch_hbm.at[pl.ds(b * _P + p0, _RPW)], patch_v)
        pltpu.sync_copy(vb_hbm.at[wid], var_v)
        row_base = var_v[...] * _S
        per_chunk = _CH // _LANES
        for j in range(_RPW // _LANES):
            sl = patch_v[pl.ds(j * _LANES, _LANES)]
            idx_v[j // per_chunk, pl.ds((j % per_chunk) * _LANES, _LANES)] = (
                row_base + sl)
        copies = [
            pltpu.make_async_copy(table_hbm.at[idx_v.at[c]], rows_v.at[c], sem)
            for c in range(_NCH)
        ]
        for cp in copies:
            cp.start()
        for cp in copies:
            cp.wait()
        pltpu.sync_copy(rows_v, out_hbm.at[wid])

    return gather_kernel


def kernel(var_indices, patch_idx, mg_embedding):
    table = mg_embedding.reshape(_NVAR * _S, _C)
    var_flat = var_indices.reshape(-1).astype(jnp.int32)
    vb = jnp.broadcast_to(
        jnp.repeat(var_flat, _NW // (_B * _V))[:, None], (_NW, _LANES))
    patch_flat = patch_idx.reshape(-1).astype(jnp.int32)
    out = _make_sc_gather()(vb, patch_flat, table)
    return out.reshape(_B, _V, 1, _P, _C)


# speedup vs baseline: 1.1133x; 1.1133x over previous
import functools

import jax
import jax.numpy as jnp
from jax import lax
from jax.experimental import pallas as pl
from jax.experimental.pallas import tpu as pltpu
from jax.experimental.pallas import tpu_sc as plsc

_B = 2
_V = 4
_P = 2048
_S = 49152
_C = 64
_NVAR = 8

_NW = 32
_RPW = (_B * _V * _P) // _NW
_CH = 128
_NCH = _RPW // _CH
_LANES = 16


def _make_sc_gather():
    info = plsc.get_sparse_core_info()
    nc = info.num_cores

    mesh = plsc.VectorSubcoreMesh(core_axis_name="c", subcore_axis_name="s")

    @functools.partial(
        pl.kernel,
        mesh=mesh,
        compiler_params=pltpu.CompilerParams(use_tc_tiling_on_sc=False),
        out_type=jax.ShapeDtypeStruct((_NW, _NCH, _CH, _C), jnp.float32),
        scratch_types=[
            pltpu.VMEM((_RPW,), jnp.int32),
            pltpu.VMEM((_LANES,), jnp.int32),
            pltpu.VMEM((_NCH, _CH), jnp.int32),
            pltpu.VMEM((_NCH, _CH, _C), jnp.float32),
            pltpu.SemaphoreType.DMA,
        ],
    )
    def gather_kernel(vb_hbm, patch_hbm, table_hbm, out_hbm,
                      patch_v, var_v, idx_v, rows_v, sem):
        wid = lax.axis_index("s") * nc + lax.axis_index("c")
        b = wid // (_NW // _B)
        p0 = (wid % (_P // _RPW)) * _RPW
        pltpu.sync_copy(patch_hbm.at[pl.ds(b * _P + p0, _RPW)], patch_v)
        pltpu.sync_copy(vb_hbm.at[wid], var_v)
        row_base = var_v[...] * _S
        per_chunk = _CH // _LANES
        for j in range(_RPW // _LANES):
            sl = patch_v[pl.ds(j * _LANES, _LANES)]
            idx_v[j // per_chunk, pl.ds((j % per_chunk) * _LANES, _LANES)] = (
                row_base + sl)
        copies = [
            pltpu.make_async_copy(table_hbm.at[idx_v.at[c]], rows_v.at[c], sem)
            for c in range(_NCH)
        ]
        for cp in copies:
            cp.start()
        for cp in copies:
            cp.wait()
        pltpu.sync_copy(rows_v, out_hbm.at[wid])

    return gather_kernel


def kernel(var_indices, patch_idx, mg_embedding):
    table = mg_embedding.reshape(_NVAR * _S, _C)
    var_flat = var_indices.reshape(-1).astype(jnp.int32)
    vb = jnp.broadcast_to(
        jnp.repeat(var_flat, _NW // (_B * _V))[:, None], (_NW, _LANES))
    patch_flat = patch_idx.reshape(-1).astype(jnp.int32)
    out = _make_sc_gather()(vb, patch_flat, table)
    return out.reshape(_B, _V, 1, _P, _C)
